# Initial kernel scaffold; baseline (speedup 1.0000x reference)
import jax
import jax.numpy as jnp
from jax.experimental import pallas as pl


def kernel(x, edge_index, edge_weight, W_mlp1, b_mlp1, W_mlp2, b_mlp2, W_z, b_z, W_r, b_r, W_h, b_h, W_lin, b_lin):
    raise NotImplementedError("write your pallas kernel here")


# speedup vs baseline: 4.7536x; 4.7536x over previous
import functools

import jax
import jax.numpy as jnp
from jax import lax
from jax.experimental import pallas as pl
from jax.experimental.pallas import tpu as pltpu
from jax.experimental.pallas import tpu_sc as plsc

N_ = 10000
E_ = 320000
F_ = 128
O_ = 16
HID_ = 100

NP_ = 10240
EP_ = 327680
ECH_ = EP_ // 128
NCH16 = ECH_ // 16
ROWS_ = NP_ // 16
RB_ = 10
BR_ = N_ // RB_


def _full(shape):
    return pl.BlockSpec(shape, lambda i: (0,) * len(shape))


def _rows(w):
    return pl.BlockSpec((BR_, w), lambda i: (i, 0))


def _tc1_body(x_r, w1_r, b1_r, w2_r, b2_r, wx_r, wh_r, ba_r, h1_o, big_o):
    xv = x_r[...]
    h = jnp.maximum(
        jnp.dot(xv, w1_r[...], preferred_element_type=jnp.float32) + b1_r[...],
        0.0,
    )
    h1 = jnp.dot(h, w2_r[...], preferred_element_type=jnp.float32) + b2_r[...]
    h1_o[...] = h1
    big_o[...] = (
        jnp.dot(xv, wx_r[...], preferred_element_type=jnp.float32)
        + jnp.dot(h1, wh_r[...], preferred_element_type=jnp.float32)
        + ba_r[...]
    )


def _tc1(x, w1, b1, w2, b2, wx_all, wh1_all, b_all):
    return pl.pallas_call(
        _tc1_body,
        grid=(RB_,),
        in_specs=[
            _rows(F_),
            _full((F_, HID_)),
            _full((1, HID_)),
            _full((HID_, O_)),
            _full((1, O_)),
            _full((F_, 240)),
            _full((O_, 240)),
            _full((1, 240)),
        ],
        out_specs=[_rows(O_), _rows(240)],
        out_shape=[
            jax.ShapeDtypeStruct((N_, O_), jnp.float32),
            jax.ShapeDtypeStruct((N_, 240), jnp.float32),
        ],
    )(x, w1, b1, w2, b2, wx_all, wh1_all, b_all)


def _tc2_body(zr0_r, uo_r, ui_r, h1_r, hx_r, whf_r, z_o, ah_o):
    zp = zr0_r[...] + uo_r[...] + ui_r[...]
    zg = jax.nn.sigmoid(zp[:, :O_])
    rg = jax.nn.sigmoid(zp[:, O_:])
    rh = rg * h1_r[...]
    z_o[...] = zg
    ah_o[...] = hx_r[...] + jnp.dot(
        rh, whf_r[...], preferred_element_type=jnp.float32
    )


def _tc2(zr0, u_o, u_i, h1, hx, whf):
    return pl.pallas_call(
        _tc2_body,
        grid=(RB_,),
        in_specs=[
            _rows(32),
            _rows(32),
            _rows(32),
            _rows(O_),
            _rows(80),
            _full((O_, 80)),
        ],
        out_specs=[_rows(O_), _rows(80)],
        out_shape=[
            jax.ShapeDtypeStruct((N_, O_), jnp.float32),
            jax.ShapeDtypeStruct((N_, 80), jnp.float32),
        ],
    )(zr0, u_o, u_i, h1, hx, whf)


def _tc3_body(z_r, h1_r, h0_r, uo_r, ui_r, wl_r, bl_r, y_o):
    ht = jnp.tanh(h0_r[...] + uo_r[...] + ui_r[...])
    zv = z_r[...]
    hcur = zv * h1_r[...] + (1.0 - zv) * ht
    y_o[...] = (
        jnp.sum(jnp.maximum(hcur, 0.0) * wl_r[...], axis=1, keepdims=True)
        + bl_r[...]
    )


def _tc3(z, h1, h0, uh_o, uh_i, wl_row, bl):
    return pl.pallas_call(
        _tc3_body,
        grid=(RB_,),
        in_specs=[
            _rows(O_),
            _rows(O_),
            _rows(O_),
            _rows(O_),
            _rows(O_),
            _full((1, O_)),
            _full((1, 1)),
        ],
        out_specs=[_rows(1)],
        out_shape=[jax.ShapeDtypeStruct((N_, 1), jnp.float32)],
    )(z, h1, h0, uh_o, uh_i, wl_row, bl)


def _norms_body(rowr, colr, wr, no_out, ni_out,
                rowv, colv, wv, reco_v, reci_v, dbuf, nob, nib,
                deg_o, deg_i, sem):
    s = lax.axis_index("s")
    n0 = s * ROWS_

    def zb(i, _):
        dbuf[pl.ds(i * 16, 16)] = jnp.zeros((16,), jnp.float32)
        return 0

    lax.fori_loop(0, ROWS_ // 16, zb, 0)
    pltpu.sync_copy(dbuf, deg_o.at[pl.ds(n0, ROWS_)])
    pltpu.sync_copy(dbuf, deg_i.at[pl.ds(n0, ROWS_)])

    pltpu.sync_copy(rowr.at[pl.ds(s * NCH16, NCH16)], rowv)
    pltpu.sync_copy(colr.at[pl.ds(s * NCH16, NCH16)], colv)
    pltpu.sync_copy(wr.at[pl.ds(s * NCH16, NCH16)], wv)
    plsc.subcore_barrier()

    def deg_step(j, _):
        pltpu.sync_copy(wv.at[j], deg_o.at[rowv.at[j]], add=True)
        pltpu.sync_copy(wv.at[j], deg_i.at[colv.at[j]], add=True)
        return 0

    lax.fori_loop(0, NCH16, deg_step, 0)
    plsc.subcore_barrier()

    def recip(deg):
        pltpu.sync_copy(deg.at[pl.ds(n0, ROWS_)], dbuf)

        def rstep(i, _):
            sl = pl.ds(i * 16, 16)
            dbuf[sl] = 1.0 / jnp.maximum(dbuf[sl], 1e-12)
            return 0

        lax.fori_loop(0, ROWS_ // 16, rstep, 0)
        pltpu.sync_copy(dbuf, deg.at[pl.ds(n0, ROWS_)])

    recip(deg_o)
    recip(deg_i)
    plsc.subcore_barrier()

    pltpu.sync_copy(deg_o, reco_v)
    pltpu.sync_copy(deg_i, reci_v)

    def norm_step(j, _):
        for v in range(8):
            sl = pl.ds(v * 16, 16)
            wvec = wv[j, sl]
            nob[sl] = plsc.load_gather(reco_v, [rowv[j, sl]]) * wvec
            nib[sl] = plsc.load_gather(reci_v, [colv[j, sl]]) * wvec
        pltpu.sync_copy(nob, no_out.at[s * NCH16 + j])
        pltpu.sync_copy(nib, ni_out.at[s * NCH16 + j])
        return 0

    lax.fori_loop(0, NCH16, norm_step, 0)


def _norms(rowp, colp, wp):
    mesh = plsc.VectorSubcoreMesh(core_axis_name="c", subcore_axis_name="s")
    f = functools.partial(
        pl.kernel,
        out_type=[
            jax.ShapeDtypeStruct((ECH_, 128), jnp.float32),
            jax.ShapeDtypeStruct((ECH_, 128), jnp.float32),
        ],
        mesh=mesh,
        scratch_types=[
            pltpu.VMEM((NCH16, 128), jnp.int32),
            pltpu.VMEM((NCH16, 128), jnp.int32),
            pltpu.VMEM((NCH16, 128), jnp.float32),
            pltpu.VMEM((NP_,), jnp.float32),
            pltpu.VMEM((NP_,), jnp.float32),
            pltpu.VMEM((ROWS_,), jnp.float32),
            pltpu.VMEM((128,), jnp.float32),
            pltpu.VMEM((128,), jnp.float32),
            pltpu.VMEM_SHARED((NP_,), jnp.float32),
            pltpu.VMEM_SHARED((NP_,), jnp.float32),
            pltpu.SemaphoreType.DMA,
        ],
        compiler_params=pltpu.CompilerParams(needs_layout_passes=False, use_tc_tiling_on_sc=False),
    )(_norms_body)
    return f(rowp, colp, wp)


def _make_prop_body(w):
    nh = w // 16

    def body(colr, rowr, no_r, ni_r, a_o1, a_o2, a_i1, a_i2,
             u_o, u_i, t_o, t_i,
             gidx, sidx, nrm, buf, nb1, nb2, acc, sem):
        c = lax.axis_index("c")
        s = lax.axis_index("s")
        n0 = s * ROWS_

        def zero_acc():
            def zb(i, _):
                for h in range(nh):
                    buf[i, pl.ds(h * 16, 16)] = jnp.zeros((16,), jnp.float32)
                return 0

            lax.fori_loop(0, 128, zb, 0)
            for k in range(ROWS_ // 128):
                pltpu.sync_copy(buf, acc.at[pl.ds(n0 + k * 128, 128)])

        zero_acc()

        iota16 = lax.iota(jnp.int32, 16)

        def hop(tbl):
            def chunk(j, _):
                pltpu.async_copy(tbl.at[gidx.at[j]], buf, sem).wait()

                def scale(v, _):
                    eidx = iota16 + v * 16
                    nv = nrm[j, pl.ds(v * 16, 16)]
                    for f in range(w):
                        cf = jnp.full((16,), f, jnp.int32)
                        vals = plsc.load_gather(buf, [eidx, cf]) * nv
                        plsc.store_scatter(buf, [eidx, cf], vals)
                    return 0

                lax.fori_loop(0, 8, scale, 0)
                pltpu.sync_copy(buf, acc.at[sidx.at[j]], add=True)
                return 0

            lax.fori_loop(0, NCH16, chunk, 0)

        def run_dir(gref, sref, nref, tbl2, a1, tout, uout):
            pltpu.sync_copy(gref.at[pl.ds(s * NCH16, NCH16)], gidx)
            pltpu.sync_copy(sref.at[pl.ds(s * NCH16, NCH16)], sidx)
            pltpu.sync_copy(nref.at[pl.ds(s * NCH16, NCH16)], nrm)
            plsc.subcore_barrier()
            hop(tbl2)
            plsc.subcore_barrier()
            pltpu.sync_copy(acc.at[pl.ds(n0, ROWS_)], nb1)
            pltpu.sync_copy(a1.at[pl.ds(n0, ROWS_)], nb2)

            def nstep(i, _):
                for h in range(nh):
                    sl = pl.ds(h * 16, 16)
                    nb1[i, sl] = nb2[i, sl] + 2.0 * nb1[i, sl]
                return 0

            lax.fori_loop(0, ROWS_, nstep, 0)
            pltpu.sync_copy(nb1, tout.at[pl.ds(n0, ROWS_)])
            plsc.subcore_barrier()
            zero_acc()
            plsc.subcore_barrier()
            hop(tout)
            plsc.subcore_barrier()
            pltpu.sync_copy(acc.at[pl.ds(n0, ROWS_)], nb1)
            pltpu.sync_copy(nb1, uout.at[pl.ds(n0, ROWS_)])

        @pl.when(c == 0)
        def _():
            run_dir(colr, rowr, no_r, a_o2, a_o1, t_o, u_o)

        @pl.when(c == 1)
        def _():
            run_dir(rowr, colr, ni_r, a_i2, a_i1, t_i, u_i)

    return body


def _prop(w, colp, rowp, norm_o, norm_i, a_o1, a_o2, a_i1, a_i2):
    mesh = plsc.VectorSubcoreMesh(core_axis_name="c", subcore_axis_name="s")
    f = functools.partial(
        pl.kernel,
        out_type=[jax.ShapeDtypeStruct((NP_, w), jnp.float32)] * 4,
        mesh=mesh,
        scratch_types=[
            pltpu.VMEM((NCH16, 128), jnp.int32),
            pltpu.VMEM((NCH16, 128), jnp.int32),
            pltpu.VMEM((NCH16, 128), jnp.float32),
            pltpu.VMEM((128, w), jnp.float32),
            pltpu.VMEM((ROWS_, w), jnp.float32),
            pltpu.VMEM((ROWS_, w), jnp.float32),
            pltpu.VMEM_SHARED((NP_, w), jnp.float32),
            pltpu.SemaphoreType.DMA,
        ],
        compiler_params=pltpu.CompilerParams(needs_layout_passes=False, use_tc_tiling_on_sc=False),
    )(_make_prop_body(w))
    u_o, u_i, _t_o, _t_i = f(colp, rowp, norm_o, norm_i, a_o1, a_o2, a_i1, a_i2)
    return u_o, u_i


def _padn(a):
    return jnp.pad(a, ((0, NP_ - N_), (0, 0)))


def kernel(x, edge_index, edge_weight, W_mlp1, b_mlp1, W_mlp2, b_mlp2,
           W_z, b_z, W_r, b_r, W_h, b_h, W_lin, b_lin):
    pad_e = EP_ - E_
    row = edge_index[0]
    col = edge_index[1]
    rowp = jnp.concatenate(
        [row, jnp.full((pad_e,), N_, jnp.int32)]).reshape(ECH_, 128)
    colp = jnp.concatenate(
        [col, jnp.full((pad_e,), N_, jnp.int32)]).reshape(ECH_, 128)
    wp = jnp.concatenate(
        [edge_weight, jnp.zeros((pad_e,), jnp.float32)]).reshape(ECH_, 128)

    def lump(wm):
        return wm[0, 0] + wm[1, 0] - wm[0, 2] - wm[1, 2]

    def zr(d, k):
        return jnp.concatenate([W_z[d, k], W_r[d, k]], axis=1)

    wbig = jnp.concatenate(
        [zr(0, 1), zr(0, 2), zr(1, 1), zr(1, 2),
         jnp.concatenate([lump(W_z), lump(W_r)], axis=1)], axis=1)
    wxh = jnp.concatenate(
        [W_h[0, 1][:F_], W_h[0, 2][:F_], W_h[1, 1][:F_], W_h[1, 2][:F_],
         lump(W_h)[:F_]], axis=1)
    wx_all = jnp.concatenate([wbig[:F_], wxh], axis=1)
    wh1_all = jnp.concatenate(
        [wbig[F_:], jnp.zeros((O_, 80), jnp.float32)], axis=1)
    b_all = jnp.concatenate(
        [jnp.zeros((128,), jnp.float32), b_z, b_r,
         jnp.zeros((64,), jnp.float32), b_h]).reshape(1, 240)
    whf = jnp.concatenate(
        [W_h[0, 1][F_:], W_h[0, 2][F_:], W_h[1, 1][F_:], W_h[1, 2][F_:],
         lump(W_h)[F_:]], axis=1)

    h1, big = _tc1(x, W_mlp1, b_mlp1.reshape(1, HID_), W_mlp2,
                   b_mlp2.reshape(1, O_), wx_all, wh1_all, b_all)

    no2d, ni2d = _norms(rowp, colp, wp)
    norm_out = no2d.reshape(-1)[:E_]
    norm_in = ni2d.reshape(-1)[:E_]

    uzr_o, uzr_i = _prop(
        32, colp, rowp, no2d, ni2d,
        _padn(big[:, 0:32]), _padn(big[:, 32:64]),
        _padn(big[:, 64:96]), _padn(big[:, 96:128]))

    z, ah = _tc2(big[:, 128:160], uzr_o[:N_], uzr_i[:N_], h1,
                 big[:, 160:240], whf)

    uh_o, uh_i = _prop(
        16, colp, rowp, no2d, ni2d,
        _padn(ah[:, 0:16]), _padn(ah[:, 16:32]),
        _padn(ah[:, 32:48]), _padn(ah[:, 48:64]))

    y = _tc3(z, h1, ah[:, 64:80], uh_o[:N_], uh_i[:N_],
             W_lin.reshape(1, O_), b_lin.reshape(1, 1))[0]

    a_out = jnp.stack([norm_out, norm_in], axis=0)
    return (y, a_out)
